# baseline (speedup 1.0000x reference)
import functools

import numpy as np
import jax
import jax.numpy as jnp
from jax.experimental import pallas as pl
from jax.experimental.pallas import tpu as pltpu
from jax.experimental.pallas import tpu_sc as plsc

_D_MODEL = 512
_PE_LEN = 2048
_SQRT_D = float(np.sqrt(float(_D_MODEL)))

_NUM_CORES = 2
_NUM_SUBCORES = 16
_NUM_WORKERS = _NUM_CORES * _NUM_SUBCORES

_LANES = 16

_CHUNK = 32
_PAIR = 2 * _CHUNK


def _pos_encoding_np(length: int, depth: int) -> np.ndarray:
    half = depth / 2
    positions = np.arange(length)[:, np.newaxis]
    depths = np.arange(half)[np.newaxis, :] / half
    angle_rates = 1.0 / (10000.0 ** depths)
    angle_rads = positions * angle_rates
    return np.concatenate(
        [np.sin(angle_rads), np.cos(angle_rads)], axis=-1
    ).astype(np.float32)


_PE_NP = _pos_encoding_np(_PE_LEN, _D_MODEL)


def _sc_fused(table, idx_flat, pe, length):
    n = idx_flat.shape[0]
    d = table.shape[1]
    b_per_w = n // _NUM_WORKERS
    n_pairs = b_per_w // _PAIR
    mesh = plsc.VectorSubcoreMesh(core_axis_name="c", subcore_axis_name="s")

    @functools.partial(
        pl.kernel,
        out_type=jax.ShapeDtypeStruct((n, d), table.dtype),
        mesh=mesh,
        scratch_types=(
            [pltpu.VMEM((b_per_w,), jnp.int32)]
            + [pltpu.VMEM((_PAIR, d), jnp.float32) for _ in range(2)]
            + [pltpu.VMEM((_PAIR, d), jnp.float32)]
            + [pltpu.SemaphoreType.DMA for _ in range(2)]
            + [pltpu.SemaphoreType.DMA]
            + [pltpu.SemaphoreType.DMA for _ in range(n_pairs)]
        ),
    )
    def fused_kernel(
        tbl_hbm, i_hbm, pe_hbm, o_hbm, idx_v, rows0, rows1, pe_v, *sems
    ):
        rows = (rows0, rows1)
        gsem = sems[:2]
        psem = sems[2]
        wsem = sems[3:]
        wid = jax.lax.axis_index("s") * _NUM_CORES + jax.lax.axis_index("c")
        base = wid * b_per_w
        pltpu.sync_copy(i_hbm.at[pl.ds(base, b_per_w)], idx_v)

        def gather_desc(p, half):
            buf = p % 2
            return pltpu.make_async_copy(
                tbl_hbm.at[
                    idx_v.at[pl.ds(p * _PAIR + half * _CHUNK, _CHUNK)]
                ],
                rows[buf].at[pl.ds(half * _CHUNK, _CHUNK)],
                gsem[buf],
            )

        def pe_desc(p):
            off = jax.lax.rem(base + p * _PAIR, length)
            return pltpu.make_async_copy(
                pe_hbm.at[pl.ds(off, _PAIR)], pe_v, psem
            )

        def write_desc(p):
            return pltpu.make_async_copy(
                rows[p % 2],
                o_hbm.at[pl.ds(base + p * _PAIR, _PAIR)],
                wsem[p],
            )

        pe_desc(0).start()
        for half in range(2):
            gather_desc(0, half).start()
        if n_pairs > 1:
            for half in range(2):
                gather_desc(1, half).start()
        for p in range(n_pairs):
            buf = p % 2
            if p >= 1 and p + 1 < n_pairs:
                write_desc(p - 1).wait()
                for half in range(2):
                    gather_desc(p + 1, half).start()
            for half in range(2):
                gather_desc(p, half).wait()
            pe_desc(p).wait()

            rbuf = rows[buf]

            @pl.loop(0, _PAIR)
            def _(r, rbuf=rbuf):
                for k in range(0, d, _LANES):
                    sl = (r, pl.ds(k, _LANES))
                    rbuf[sl] = rbuf[sl] * _SQRT_D + pe_v[sl]

            if p + 1 < n_pairs:
                pe_desc(p + 1).start()
            write_desc(p).start()
        for p in range(max(0, n_pairs - 2), n_pairs):
            write_desc(p).wait()

    return fused_kernel(table, idx_flat, pe)


@jax.jit
def kernel(x, table):
    batch, length = x.shape
    idx = x.reshape(batch * length).astype(jnp.int32)
    pe = jnp.asarray(_PE_NP[:length])
    out = _sc_fused(table, idx, pe, length)
    return out.reshape(batch, length, table.shape[1])


# speedup vs baseline: 1.1304x; 1.1304x over previous
import functools

import numpy as np
import jax
import jax.numpy as jnp
from jax.experimental import pallas as pl
from jax.experimental.pallas import tpu as pltpu
from jax.experimental.pallas import tpu_sc as plsc

_D_MODEL = 512
_PE_LEN = 2048
_SQRT_D = float(np.sqrt(float(_D_MODEL)))

_NUM_CORES = 2
_NUM_SUBCORES = 16
_NUM_WORKERS = _NUM_CORES * _NUM_SUBCORES

_LANES = 16

_CHUNK = 32
_NBUF = 3


def _pos_encoding_np(length: int, depth: int) -> np.ndarray:
    half = depth / 2
    positions = np.arange(length)[:, np.newaxis]
    depths = np.arange(half)[np.newaxis, :] / half
    angle_rates = 1.0 / (10000.0 ** depths)
    angle_rads = positions * angle_rates
    return np.concatenate(
        [np.sin(angle_rads), np.cos(angle_rads)], axis=-1
    ).astype(np.float32)


_PE_NP = _pos_encoding_np(_PE_LEN, _D_MODEL)


def _sc_fused(table, idx_flat, pe, length):
    n = idx_flat.shape[0]
    d = table.shape[1]
    b_per_w = n // _NUM_WORKERS
    n_chunks = b_per_w // _CHUNK
    mesh = plsc.VectorSubcoreMesh(core_axis_name="c", subcore_axis_name="s")

    @functools.partial(
        pl.kernel,
        out_type=jax.ShapeDtypeStruct((n, d), table.dtype),
        mesh=mesh,
        scratch_types=(
            [pltpu.VMEM((b_per_w,), jnp.int32)]
            + [pltpu.VMEM((_CHUNK, d), jnp.float32) for _ in range(_NBUF)]
            + [pltpu.VMEM((_CHUNK, d), jnp.float32) for _ in range(_NBUF)]
            + [pltpu.SemaphoreType.DMA for _ in range(_NBUF)]
            + [pltpu.SemaphoreType.DMA for _ in range(_NBUF)]
            + [pltpu.SemaphoreType.DMA for _ in range(n_chunks)]
        ),
    )
    def fused_kernel(tbl_hbm, i_hbm, pe_hbm, o_hbm, idx_v, *scratch):
        rows = scratch[:_NBUF]
        peb = scratch[_NBUF : 2 * _NBUF]
        gsem = scratch[2 * _NBUF : 3 * _NBUF]
        psem = scratch[3 * _NBUF : 4 * _NBUF]
        wsem = scratch[4 * _NBUF :]
        wid = jax.lax.axis_index("s") * _NUM_CORES + jax.lax.axis_index("c")
        base = wid * b_per_w
        pltpu.sync_copy(i_hbm.at[pl.ds(base, b_per_w)], idx_v)

        def gather_desc(c, buf):
            return pltpu.make_async_copy(
                tbl_hbm.at[idx_v.at[pl.ds(c * _CHUNK, _CHUNK)]],
                rows[buf],
                gsem[buf],
            )

        def pe_desc(c, buf):
            off = jax.lax.rem(base + c * _CHUNK, length)
            return pltpu.make_async_copy(
                pe_hbm.at[pl.ds(off, _CHUNK)], peb[buf], psem[buf]
            )

        def write_desc(c, buf):
            return pltpu.make_async_copy(
                rows[buf],
                o_hbm.at[pl.ds(base + c * _CHUNK, _CHUNK)],
                wsem[c],
            )

        for c in range(min(_NBUF, n_chunks)):
            gather_desc(c, c).start()
            pe_desc(c, c).start()
        for c in range(n_chunks):
            buf = c % _NBUF
            prev = c - 1
            nxt = prev + _NBUF
            if prev >= 0 and nxt < n_chunks:
                pbi = prev % _NBUF
                write_desc(prev, pbi).wait()
                gather_desc(nxt, pbi).start()
                pe_desc(nxt, pbi).start()
            gather_desc(c, buf).wait()
            pe_desc(c, buf).wait()

            rbuf, pbuf = rows[buf], peb[buf]

            @pl.loop(0, _CHUNK)
            def _(r, rbuf=rbuf, pbuf=pbuf):
                for k in range(0, d, _LANES):
                    sl = (r, pl.ds(k, _LANES))
                    rbuf[sl] = rbuf[sl] * _SQRT_D + pbuf[sl]

            write_desc(c, buf).start()
        for c in range(max(0, n_chunks - _NBUF), n_chunks):
            write_desc(c, c % _NBUF).wait()

    return fused_kernel(table, idx_flat, pe)


@jax.jit
def kernel(x, table):
    batch, length = x.shape
    idx = x.reshape(batch * length).astype(jnp.int32)
    pe = jnp.asarray(_PE_NP[:length])
    out = _sc_fused(table, idx, pe, length)
    return out.reshape(batch, length, table.shape[1])
